# baseline (speedup 1.0000x reference)
import jax
import jax.numpy as jnp
from jax import lax
from jax.experimental import pallas as pl
from jax.experimental.pallas import tpu as pltpu
from jax.experimental.pallas import tpu_sc as plsc

_CH = 32
_NCH = 1024 // _CH
_NW = 32


def _sc_body(x_hbm, kk_hbm, tok_hbm, out_hbm,
             kk_v, mask_buf, buf_a, buf_b, buf_m,
             sem_fill, sem_mix, sem_out_a, sem_out_b, sem_in_a, sem_in_b):
    c = lax.axis_index("c")
    s = lax.axis_index("s")
    w = s * 2 + c

    pltpu.sync_copy(kk_hbm, kk_v.at[pl.ds(0, 16)])

    for i in range(_CH):
        pltpu.async_copy(tok_hbm, mask_buf.at[i], sem_fill)
    pltpu.make_async_copy(x_hbm.at[0, 0, pl.ds(0, _CH)], mask_buf,
                          sem_fill).wait()

    def task(j, carry):
        b = j // 16
        t = j % 16
        kk = kk_v[pl.ds(t, 16)][0]
        ci = lax.rem(w + j, _NW)
        lo = ci * _CH
        hi = lo + _CH
        sl = pl.ds(lo, _CH)

        is_copy = kk >= hi
        is_fill = kk <= lo
        is_mixed = jnp.logical_and(kk > lo, kk < hi)

        @pl.when(is_fill)
        def _fill():
            pltpu.async_copy(mask_buf, out_hbm.at[b, t, sl], sem_fill)

        cnt = carry[2]
        pj = carry[3]
        plo = pl.multiple_of(carry[4], _CH)

        @pl.when(jnp.logical_and(is_copy, lax.rem(cnt, 2) == 0))
        def _copy_a():
            @pl.when(cnt >= 2)
            def _():
                pltpu.make_async_copy(buf_a,
                                      out_hbm.at[0, 0, pl.ds(0, _CH)],
                                      sem_out_a).wait()
            pltpu.async_copy(x_hbm.at[b, t, sl], buf_a, sem_in_a)

            @pl.when(cnt >= 1)
            def _():
                pltpu.make_async_copy(x_hbm.at[0, 0, pl.ds(0, _CH)],
                                      buf_b, sem_in_b).wait()
                pltpu.async_copy(buf_b,
                                 out_hbm.at[pj // 16, pj % 16,
                                            pl.ds(plo, _CH)], sem_out_b)

        @pl.when(jnp.logical_and(is_copy, lax.rem(cnt, 2) == 1))
        def _copy_b():
            @pl.when(cnt >= 2)
            def _():
                pltpu.make_async_copy(buf_b,
                                      out_hbm.at[0, 0, pl.ds(0, _CH)],
                                      sem_out_b).wait()
            pltpu.async_copy(x_hbm.at[b, t, sl], buf_b, sem_in_b)

            pltpu.make_async_copy(x_hbm.at[0, 0, pl.ds(0, _CH)],
                                  buf_a, sem_in_a).wait()
            pltpu.async_copy(buf_a,
                             out_hbm.at[pj // 16, pj % 16,
                                        pl.ds(plo, _CH)], sem_out_a)

        @pl.when(is_mixed)
        def _mixed():
            m_used = carry[1]

            @pl.when(m_used > 0)
            def _():
                pltpu.make_async_copy(buf_m,
                                      out_hbm.at[0, 0, pl.ds(0, _CH)],
                                      sem_mix).wait()

            pltpu.sync_copy(x_hbm.at[b, t, sl], buf_m)
            r = kk - lo

            def row(n, rc):
                @pl.when(n < r)
                def _():
                    pltpu.async_copy(buf_m.at[n], out_hbm.at[b, t, lo + n],
                                     sem_mix)

                @pl.when(n >= r)
                def _():
                    pltpu.async_copy(mask_buf.at[0], out_hbm.at[b, t, lo + n],
                                     sem_mix)

                return rc

            lax.fori_loop(0, _CH, row, 0)

        one = jnp.int32(1)
        zero = jnp.int32(0)
        nfill = carry[0] + jnp.where(is_fill, one, zero)
        m_used = carry[1] + jnp.where(is_mixed, one, zero)
        pj = jnp.where(is_copy, j, pj).astype(pj.dtype)
        plo = jnp.where(is_copy, lo, plo).astype(plo.dtype)
        cnt = cnt + jnp.where(is_copy, one, zero)
        return nfill, m_used, cnt, pj, plo

    nfill, m_used, cnt, pj, plo = lax.fori_loop(
        0, _NW, task,
        (jnp.int32(0), jnp.int32(0), jnp.int32(0), jnp.int32(0), jnp.int32(0)))

    plo = pl.multiple_of(plo, _CH)

    @pl.when(cnt >= 1)
    def _():
        @pl.when(lax.rem(cnt, 2) == 1)
        def _():
            pltpu.make_async_copy(x_hbm.at[0, 0, pl.ds(0, _CH)],
                                  buf_a, sem_in_a).wait()
            pltpu.async_copy(buf_a,
                             out_hbm.at[pj // 16, pj % 16, pl.ds(plo, _CH)],
                             sem_out_a)

        @pl.when(lax.rem(cnt, 2) == 0)
        def _():
            pltpu.make_async_copy(x_hbm.at[0, 0, pl.ds(0, _CH)],
                                  buf_b, sem_in_b).wait()
            pltpu.async_copy(buf_b,
                             out_hbm.at[pj // 16, pj % 16, pl.ds(plo, _CH)],
                             sem_out_b)

    def drain_fill(i, carry):
        pltpu.make_async_copy(mask_buf,
                              out_hbm.at[0, 0, pl.ds(0, _CH)], sem_fill).wait()
        return carry

    lax.fori_loop(0, nfill, drain_fill, 0)

    @pl.when(m_used > 0)
    def _():
        pltpu.make_async_copy(buf_m,
                              out_hbm.at[0, 0, pl.ds(0, _CH)], sem_mix).wait()

    @pl.when(cnt >= 1)
    def _():
        @pl.when(lax.rem(cnt, 2) == 1)
        def _():
            pltpu.make_async_copy(buf_a,
                                  out_hbm.at[0, 0, pl.ds(0, _CH)],
                                  sem_out_a).wait()

        @pl.when(lax.rem(cnt, 2) == 0)
        def _():
            pltpu.make_async_copy(buf_b,
                                  out_hbm.at[0, 0, pl.ds(0, _CH)],
                                  sem_out_b).wait()

    @pl.when(cnt >= 2)
    def _():
        @pl.when(lax.rem(cnt, 2) == 1)
        def _():
            pltpu.make_async_copy(buf_b,
                                  out_hbm.at[0, 0, pl.ds(0, _CH)],
                                  sem_out_b).wait()

        @pl.when(lax.rem(cnt, 2) == 0)
        def _():
            pltpu.make_async_copy(buf_a,
                                  out_hbm.at[0, 0, pl.ds(0, _CH)],
                                  sem_out_a).wait()


def kernel(x, keep_k, mask_token):
    D = x.shape[-1]
    mesh = plsc.VectorSubcoreMesh(core_axis_name="c", subcore_axis_name="s",
                                  num_cores=2, num_subcores=16)
    f = pl.kernel(
        _sc_body,
        out_type=jax.ShapeDtypeStruct(x.shape, x.dtype),
        mesh=mesh,
        scratch_types=[
            pltpu.VMEM((32,), jnp.int32),
            pltpu.VMEM((_CH, D), jnp.float32),
            pltpu.VMEM((_CH, D), jnp.float32),
            pltpu.VMEM((_CH, D), jnp.float32),
            pltpu.VMEM((_CH, D), jnp.float32),
            pltpu.SemaphoreType.DMA,
            pltpu.SemaphoreType.DMA,
            pltpu.SemaphoreType.DMA,
            pltpu.SemaphoreType.DMA,
            pltpu.SemaphoreType.DMA,
            pltpu.SemaphoreType.DMA,
        ],
    )
    return f(x, keep_k.astype(jnp.int32), mask_token)
